# Initial kernel scaffold; baseline (speedup 1.0000x reference)
import jax
import jax.numpy as jnp
from jax.experimental import pallas as pl


def kernel(x, Wr, br, W1, b1, W2, b2, gamma, beta):
    raise NotImplementedError("write your pallas kernel here")


# speedup vs baseline: 2.2117x; 2.2117x over previous
import jax
import jax.numpy as jnp
from jax.experimental import pallas as pl
from jax.experimental.pallas import tpu as pltpu

_B, _T, _D = 2, 2048, 1024
_E, _TOPK, _HID = 8, 2, 2048
_N = _B * _T
_TM = 256
_NT = _N // _TM


def _router_kernel(x_ref, wr_ref, br_ref, noise_ref, w_ref):
    logits = jnp.dot(x_ref[...], wr_ref[...], preferred_element_type=jnp.float32)
    logits = logits + br_ref[...] + noise_ref[...]
    m = jnp.max(logits, axis=-1, keepdims=True)
    ex = jnp.exp(logits - m)
    sm = ex / jnp.sum(ex, axis=-1, keepdims=True)
    e_iota = jax.lax.broadcasted_iota(jnp.int32, sm.shape, 1)
    m1 = jnp.max(sm, axis=-1, keepdims=True)
    i1 = jnp.min(jnp.where(sm == m1, e_iota, _E), axis=-1, keepdims=True)
    sm2 = jnp.where(e_iota == i1, -jnp.inf, sm)
    m2 = jnp.max(sm2, axis=-1, keepdims=True)
    i2 = jnp.min(jnp.where(sm2 == m2, e_iota, _E), axis=-1, keepdims=True)
    w = jnp.where(e_iota == i1, m1, 0.0) + jnp.where(e_iota == i2, m2, 0.0)
    w_ref[...] = w


_SQRT_HALF = 0.7071067811865476


def _gelu(v):
    return 0.5 * v * (1.0 + jax.lax.erf(v * _SQRT_HALF))


def _ffn_kernel(x_ref, w_ref, w1_ref, b1_ref, w2_ref, b2_ref, gamma_ref,
                beta_ref, out_ref, acc_ref):
    e = pl.program_id(0)
    t = pl.program_id(1)
    xb = x_ref[...].astype(jnp.bfloat16)
    h = jnp.dot(xb, w1_ref[0].astype(jnp.bfloat16),
                preferred_element_type=jnp.float32)
    h = _gelu(h + b1_ref[0])
    o = jnp.dot(h.astype(jnp.bfloat16), w2_ref[0].astype(jnp.bfloat16),
                preferred_element_type=jnp.float32)
    wfull = w_ref[...]
    lane = jax.lax.broadcasted_iota(jnp.int32, wfull.shape, 1)
    wcol = jnp.sum(jnp.where(lane == e, wfull, 0.0), axis=1, keepdims=True)
    o = (o + b2_ref[0]) * wcol
    rows = pl.ds(t * _TM, _TM)

    @pl.when(e == 0)
    def _init():
        acc_ref[rows, :] = o

    @pl.when(e != 0)
    def _accum():
        acc_ref[rows, :] += o

    @pl.when(e == _E - 1)
    def _finish():
        y = acc_ref[rows, :] + x_ref[...]
        mu = jnp.mean(y, axis=-1, keepdims=True)
        yc = y - mu
        var = jnp.mean(yc * yc, axis=-1, keepdims=True)
        out_ref[...] = yc * jax.lax.rsqrt(var + 1e-5) * gamma_ref[...] + beta_ref[...]


def kernel(x, Wr, br, W1, b1, W2, b2, gamma, beta):
    xf = x.reshape(_N, _D)
    noise = jax.random.normal(jax.random.key(42), (_N, _E), jnp.float32) / 10.0

    w_comb = pl.pallas_call(
        _router_kernel,
        grid=(_NT,),
        in_specs=[
            pl.BlockSpec((_TM, _D), lambda t: (t, 0)),
            pl.BlockSpec((_D, _E), lambda t: (0, 0)),
            pl.BlockSpec((1, _E), lambda t: (0, 0)),
            pl.BlockSpec((_TM, _E), lambda t: (t, 0)),
        ],
        out_specs=pl.BlockSpec((_TM, _E), lambda t: (t, 0)),
        out_shape=jax.ShapeDtypeStruct((_N, _E), jnp.float32),
    )(xf, Wr, br.reshape(1, _E), noise)

    y = pl.pallas_call(
        _ffn_kernel,
        grid=(_E, _NT),
        in_specs=[
            pl.BlockSpec((_TM, _D), lambda e, t: (t, 0)),
            pl.BlockSpec((_TM, _E), lambda e, t: (t, 0)),
            pl.BlockSpec((1, _D, _HID), lambda e, t: (e, 0, 0)),
            pl.BlockSpec((1, 1, _HID), lambda e, t: (e, 0, 0)),
            pl.BlockSpec((1, _HID, _D), lambda e, t: (e, 0, 0)),
            pl.BlockSpec((1, 1, _D), lambda e, t: (e, 0, 0)),
            pl.BlockSpec((1, _D), lambda e, t: (0, 0)),
            pl.BlockSpec((1, _D), lambda e, t: (0, 0)),
        ],
        out_specs=pl.BlockSpec((_TM, _D), lambda e, t: (t, 0)),
        out_shape=jax.ShapeDtypeStruct((_N, _D), jnp.float32),
        scratch_shapes=[pltpu.VMEM((_N, _D), jnp.float32)],
        compiler_params=pltpu.CompilerParams(
            vmem_limit_bytes=100 * 1024 * 1024,
        ),
    )(xf, w_comb, W1, b1.reshape(_E, 1, _HID), W2, b2.reshape(_E, 1, _D),
      gamma.reshape(1, _D), beta.reshape(1, _D))

    return y.reshape(_B, _T, _D)
